# baseline (speedup 1.0000x reference)
import jax
import jax.numpy as jnp
import numpy as np
from jax.experimental import pallas as pl


def _identity_kernel(x_ref, o_ref):
    o_ref[...] = x_ref[...]


def _pallas_identity(x):
    return pl.pallas_call(
        _identity_kernel,
        out_shape=jax.ShapeDtypeStruct(x.shape, x.dtype),
    )(x)


def _fps(xyz, npoint):
    N = xyz.shape[0]

    def body(carry, _):
        dist, farthest = carry
        centroid = xyz[farthest]
        d = jnp.sum((xyz - centroid) ** 2, axis=-1)
        dist = jnp.minimum(dist, d)
        nxt = jnp.argmax(dist).astype(jnp.int32)
        return (dist, nxt), farthest

    init = (jnp.full((N,), 1e10, dtype=jnp.float32), jnp.array(0, dtype=jnp.int32))
    _, idx = jax.lax.scan(body, init, None, length=npoint)
    return idx


def _ball_group(xyz, new_xyz, radius, k):
    N = xyz.shape[0]
    sqr = jnp.sum((new_xyz[:, None, :] - xyz[None, :, :]) ** 2, axis=-1)
    idx = jnp.where(sqr > radius * radius, N, jnp.arange(N, dtype=jnp.int32)[None, :])
    idx = jnp.sort(idx, axis=-1)[:, :k]
    first = idx[:, :1]
    idx = jnp.where(idx == N, first, idx)
    return jnp.minimum(idx, N - 1)


def _set_abstraction(xyz, points, params, npoint, radius, k):
    fidx = _fps(xyz, npoint)
    new_xyz = xyz[fidx]
    gidx = _ball_group(xyz, new_xyz, radius, k)
    grouped = xyz[gidx] - new_xyz[:, None, :]
    if points is not None:
        grouped = jnp.concatenate([grouped, points[gidx]], axis=-1)
    h = grouped
    for W, b in params:
        h = jax.nn.relu(h @ W + b)
    new_points = jnp.max(h, axis=1)
    return new_xyz, new_points


def _frame(xyz, p1, p2):
    l1_xyz, l1_points = _set_abstraction(xyz, None, p1, 1024, 0.5, 16)
    l2_xyz, l2_points = _set_abstraction(l1_xyz, l1_points, p2, 1024, 1.0, 16)
    return l2_xyz, l2_points


def kernel(l0_xyz_f1, l0_xyz_f2, params_sa1_f1, params_sa2_f1, params_sa1_f2, params_sa2_f2):
    l0_xyz_f1 = _pallas_identity(l0_xyz_f1)
    f1x, f1p = jax.vmap(lambda x: _frame(x, params_sa1_f1, params_sa2_f1))(l0_xyz_f1)
    f2x, f2p = jax.vmap(lambda x: _frame(x, params_sa1_f2, params_sa2_f2))(l0_xyz_f2)
    return (f1x, f1p, f2x, f2p)


# speedup vs baseline: 9.8343x; 9.8343x over previous
import functools

import jax
import jax.numpy as jnp
from jax import lax
from jax.experimental import pallas as pl
from jax.experimental.pallas import tpu as pltpu
from jax.experimental.pallas import tpu_sc as plsc

F32 = jnp.float32
I32 = jnp.int32

_NC = 2
_NS = 16
_NW = _NC * _NS


def _fps_body(npoint, R, N, x_ref, o_ref, dist_ref):
    dist_ref[...] = jnp.full((R, 128), 1e10, F32)

    def body(step, far):
        flat = (lax.broadcasted_iota(I32, (R, 128), 0) * 128
                + lax.broadcasted_iota(I32, (R, 128), 1))
        sel = flat == far
        X = x_ref[0, 0]
        Y = x_ref[0, 1]
        Z = x_ref[0, 2]
        cx = jnp.sum(jnp.where(sel, X, 0.0))
        cy = jnp.sum(jnp.where(sel, Y, 0.0))
        cz = jnp.sum(jnp.where(sel, Z, 0.0))
        o_ref[0, 0, step] = cx
        o_ref[0, 1, step] = cy
        o_ref[0, 2, step] = cz
        dx = X - cx
        dy = Y - cy
        dz = Z - cz
        d = (dx * dx + dy * dy) + dz * dz
        nd = jnp.minimum(dist_ref[...], d)
        dist_ref[...] = nd
        m = jnp.max(nd)
        far2 = jnp.min(jnp.where(nd == m, flat, N))
        return far2

    lax.fori_loop(0, npoint, body, jnp.int32(0))


def _fps_pallas(xyz4, npoint, interpret=False):
    C, _, N = xyz4.shape
    R = N // 128
    xr = xyz4.reshape(C, 3, R, 128)
    out = pl.pallas_call(
        functools.partial(_fps_body, npoint, R, N),
        grid=(C,),
        in_specs=[pl.BlockSpec((1, 3, R, 128), lambda c: (c, 0, 0, 0))],
        out_specs=pl.BlockSpec((1, 3, npoint), lambda c: (c, 0, 0),
                               memory_space=pltpu.SMEM),
        out_shape=jax.ShapeDtypeStruct((C, 3, npoint), F32),
        scratch_shapes=[pltpu.VMEM((R, 128), F32)],
        interpret=interpret,
    )(xr)
    return out


def _bq_body(N, parts, r2, x_ref, c_ref, o_ref):
    cloud = pl.program_id(0) // parts
    X = x_ref[0, 0].reshape(1, N)
    Y = x_ref[0, 1].reshape(1, N)
    Z = x_ref[0, 2].reshape(1, N)
    cx = c_ref[0][:, 0:1]
    cy = c_ref[0][:, 1:2]
    cz = c_ref[0][:, 2:3]
    dx = X - cx
    dy = Y - cy
    dz = Z - cz
    dsq = (dx * dx + dy * dy) + dz * dz
    idx = lax.broadcasted_iota(I32, (1, N), 1)
    M = jnp.where(dsq <= r2, idx, N)
    mins = []
    for _ in range(16):
        mn = jnp.min(M, axis=1, keepdims=True)
        mins.append(mn)
        M = jnp.where(M == mn, N, M)
    first = mins[0]
    sel = [first] + [jnp.where(m == N, first, m) for m in mins[1:]]
    o_ref[0] = jnp.concatenate(sel, axis=1) + cloud * N


def _ball_query_tc(xyzT, nxr, radius, S=1024, interpret=False):
    C, _, N = xyzT.shape
    parts = S // 128
    r2 = radius * radius
    out = pl.pallas_call(
        functools.partial(_bq_body, N, parts, r2),
        grid=(C * parts,),
        in_specs=[
            pl.BlockSpec((1, 3, N), lambda g, parts=parts: (g // parts, 0, 0)),
            pl.BlockSpec((1, 128, 3), lambda g, parts=parts:
                         (g // parts, g % parts, 0)),
        ],
        out_specs=pl.BlockSpec((1, 128, 16), lambda g, parts=parts:
                               (g // parts, g % parts, 0)),
        out_shape=jax.ShapeDtypeStruct((C, S, 16), I32),
        interpret=interpret,
    )(xyzT, nxr.reshape(C, S, 3))
    return out


def _gather_rows_sc(table, gi, D):
    B = gi.shape[0]
    blk_per_w = B // (_NW * 128)
    gi2 = gi.reshape(B // 128, 128)
    mesh = plsc.VectorSubcoreMesh(core_axis_name="c", subcore_axis_name="s")

    @functools.partial(
        pl.kernel, mesh=mesh,
        out_type=jax.ShapeDtypeStruct((B, D), F32),
        scratch_types=[
            pltpu.VMEM((blk_per_w, 128), I32),
            pltpu.VMEM((128, D), F32),
            pltpu.SemaphoreType.DMA,
        ],
    )
    def g(tab_hbm, gi_hbm, out_hbm, gi_v, rows_v, sem):
        cid = lax.axis_index("c")
        sid = lax.axis_index("s")
        wid = sid * _NC + cid
        rowblk = wid * blk_per_w
        pltpu.sync_copy(gi_hbm.at[pl.ds(rowblk, blk_per_w)], gi_v)

        def chunk(j, _):
            pltpu.async_copy(tab_hbm.at[gi_v.at[j]], rows_v, sem).wait()
            pltpu.sync_copy(rows_v, out_hbm.at[pl.ds((rowblk + j) * 128, 128)])
            return 0

        lax.fori_loop(0, blk_per_w, chunk, 0)

    return g(table, gi2)


def _dot(a, b):
    return lax.dot_general(a, b, (((1,), (0,)), ((), ())),
                           precision=lax.Precision.HIGHEST,
                           preferred_element_type=F32)


def _mlp_body(cout, x_ref, c_ref, w1_ref, w1a_ref, b1_ref, w2_ref, b2_ref,
              w3_ref, b3_ref, o_ref):
    ch1 = w1_ref.shape[-1]
    D = x_ref.shape[-1]
    Cpad = jnp.concatenate([c_ref[0], jnp.zeros((128, D - 3), F32)], axis=1)
    Xr = (x_ref[0].reshape(128, 16, D) - Cpad[:, None, :]).reshape(2048, D)
    h = jnp.maximum(_dot(Xr, w1_ref[0]) + b1_ref[0], 0.0)
    h = jnp.maximum(_dot(h, w2_ref[0]) + b2_ref[0], 0.0)
    h = jnp.maximum(_dot(h, w3_ref[0]) + b3_ref[0], 0.0)
    o_ref[0] = jnp.max(h.reshape(128, 16, cout), axis=1)


def _mlp_pallas(Xg, Cb, w1, w1a, b1, w2, b2, w3, b3, cout):
    D = Xg.shape[-1]
    ch1 = w1.shape[-1]
    ch2 = w2.shape[-1]
    out = pl.pallas_call(
        functools.partial(_mlp_body, cout),
        grid=(32,),
        in_specs=[
            pl.BlockSpec((1, 2048, D), lambda g: (g, 0, 0)),
            pl.BlockSpec((1, 128, 3), lambda g: (g, 0, 0)),
            pl.BlockSpec((1, D, ch1), lambda g: (g // 16, 0, 0)),
            pl.BlockSpec((1, 3, ch1), lambda g: (g // 16, 0, 0)),
            pl.BlockSpec((1, 1, ch1), lambda g: (g // 16, 0, 0)),
            pl.BlockSpec((1, ch1, ch2), lambda g: (g // 16, 0, 0)),
            pl.BlockSpec((1, 1, ch2), lambda g: (g // 16, 0, 0)),
            pl.BlockSpec((1, ch2, cout), lambda g: (g // 16, 0, 0)),
            pl.BlockSpec((1, 1, cout), lambda g: (g // 16, 0, 0)),
        ],
        out_specs=pl.BlockSpec((1, 128, cout), lambda g: (g, 0, 0)),
        out_shape=jax.ShapeDtypeStruct((32, 128, cout), F32),
    )(Xg, Cb, w1, w1a, b1, w2, b2, w3, b3)
    return out


def _stack2(a, b):
    return jnp.stack([a, b])


def _prep_params(p_f1, p_f2, D, nrel):
    def w1full(p):
        W = p[0][0]
        pad = jnp.zeros((D - W.shape[0], W.shape[1]), F32)
        return jnp.concatenate([W, pad], axis=0)
    w1 = _stack2(w1full(p_f1), w1full(p_f2))
    w1a = _stack2(p_f1[0][0][0:nrel], p_f2[0][0][0:nrel])
    b1 = _stack2(p_f1[0][1], p_f2[0][1])[:, None, :]
    w2 = _stack2(p_f1[1][0], p_f2[1][0])
    b2 = _stack2(p_f1[1][1], p_f2[1][1])[:, None, :]
    w3 = _stack2(p_f1[2][0], p_f2[2][0])
    b3 = _stack2(p_f1[2][1], p_f2[2][1])[:, None, :]
    return w1, w1a, b1, w2, b2, w3, b3


def kernel(l0_xyz_f1, l0_xyz_f2, params_sa1_f1, params_sa2_f1, params_sa1_f2, params_sa2_f2):
    xyz = jnp.concatenate([l0_xyz_f1, l0_xyz_f2], axis=0)
    xyzT = jnp.transpose(xyz, (0, 2, 1))

    nx1T = _fps_pallas(xyzT, 1024)
    nxr1 = jnp.transpose(nx1T, (0, 2, 1))
    gi1 = _ball_query_tc(xyzT, nxr1, 0.5).reshape(65536)
    tab1 = jnp.concatenate(
        [xyz.reshape(4 * 16384, 3), jnp.zeros((4 * 16384, 125), F32)], axis=1)
    Xg1 = _gather_rows_sc(tab1, gi1, 128).reshape(32, 2048, 128)
    Cb1 = nxr1.reshape(32, 128, 3)
    w1, w1a, b1, w2, b2, w3, b3 = _prep_params(params_sa1_f1, params_sa1_f2,
                                               128, 3)
    l1p = _mlp_pallas(Xg1, Cb1, w1, w1a, b1, w2, b2, w3, b3, 64)
    l1p = l1p.reshape(4, 1024, 64)

    nx2T = _fps_pallas(nx1T, 1024)
    nxr2 = jnp.transpose(nx2T, (0, 2, 1))
    gi2 = _ball_query_tc(nx1T, nxr2, 1.0).reshape(65536)
    tab2 = jnp.concatenate(
        [nxr1.reshape(4096, 3), l1p.reshape(4096, 64),
         jnp.zeros((4096, 61), F32)],
        axis=1)
    Xg2 = _gather_rows_sc(tab2, gi2, 128).reshape(32, 2048, 128)
    Cb2 = nxr2.reshape(32, 128, 3)
    w1_, w1a_, b1_, w2_, b2_, w3_, b3_ = _prep_params(
        params_sa2_f1, params_sa2_f2, 128, 3)
    l2p = _mlp_pallas(Xg2, Cb2, w1_, w1a_, b1_, w2_, b2_, w3_, b3_, 128)
    l2p = l2p.reshape(4, 1024, 128)

    new_xyz2 = nxr2
    return (new_xyz2[0:2], l2p[0:2], new_xyz2[2:4], l2p[2:4])


# speedup vs baseline: 11.6929x; 1.1890x over previous
import functools

import jax
import jax.numpy as jnp
from jax import lax
from jax.experimental import pallas as pl
from jax.experimental.pallas import tpu as pltpu
from jax.experimental.pallas import tpu_sc as plsc

F32 = jnp.float32
I32 = jnp.int32

_NC = 2
_NS = 16
_NW = _NC * _NS


def _fps_body(C, npoint, R, N, x_ref, o_ref, dist_ref):
    dist_ref[...] = jnp.full((C, R, 128), 1e10, F32)
    lane = lax.broadcasted_iota(I32, (1, 128), 1)

    def body(step, fars):
        flat = (lax.broadcasted_iota(I32, (R, 128), 0) * 128
                + lax.broadcasted_iota(I32, (R, 128), 1))
        new_fars = []
        for c in range(C):
            far = fars[c]
            r = far // 128
            col = far % 128
            lsel = lane == col
            rx = x_ref[c, 0, pl.ds(r, 1), :]
            ry = x_ref[c, 1, pl.ds(r, 1), :]
            rz = x_ref[c, 2, pl.ds(r, 1), :]
            cx = jnp.sum(jnp.where(lsel, rx, 0.0))
            cy = jnp.sum(jnp.where(lsel, ry, 0.0))
            cz = jnp.sum(jnp.where(lsel, rz, 0.0))
            o_ref[c, 0, step] = cx
            o_ref[c, 1, step] = cy
            o_ref[c, 2, step] = cz
            dx = x_ref[c, 0] - cx
            dy = x_ref[c, 1] - cy
            dz = x_ref[c, 2] - cz
            d = (dx * dx + dy * dy) + dz * dz
            nd = jnp.minimum(dist_ref[c], d)
            dist_ref[c] = nd
            m = jnp.max(nd)
            new_fars.append(jnp.min(jnp.where(nd == m, flat, N)))
        return tuple(new_fars)

    lax.fori_loop(0, npoint, body, (jnp.int32(0),) * C)


def _fps_pallas(xyz4, npoint, interpret=False):
    C, _, N = xyz4.shape
    R = N // 128
    xr = xyz4.reshape(C, 3, R, 128)
    out = pl.pallas_call(
        functools.partial(_fps_body, C, npoint, R, N),
        grid=(1,),
        in_specs=[pl.BlockSpec((C, 3, R, 128), lambda g: (0, 0, 0, 0))],
        out_specs=pl.BlockSpec((C, 3, npoint), lambda g: (0, 0, 0),
                               memory_space=pltpu.SMEM),
        out_shape=jax.ShapeDtypeStruct((C, 3, npoint), F32),
        scratch_shapes=[pltpu.VMEM((C, R, 128), F32)],
        interpret=interpret,
    )(xr)
    return out


def _bq_body(N, parts, r2, x_ref, c_ref, o_ref):
    cloud = pl.program_id(0) // parts
    X = x_ref[0, 0].reshape(1, N)
    Y = x_ref[0, 1].reshape(1, N)
    Z = x_ref[0, 2].reshape(1, N)
    cx = c_ref[0][:, 0:1]
    cy = c_ref[0][:, 1:2]
    cz = c_ref[0][:, 2:3]
    dx = X - cx
    dy = Y - cy
    dz = Z - cz
    dsq = (dx * dx + dy * dy) + dz * dz
    idx = lax.broadcasted_iota(I32, (1, N), 1)
    M = jnp.where(dsq <= r2, idx, N)
    mins = []
    for _ in range(16):
        mn = jnp.min(M, axis=1, keepdims=True)
        mins.append(mn)
        M = jnp.where(M == mn, N, M)
    first = mins[0]
    sel = [first] + [jnp.where(m == N, first, m) for m in mins[1:]]
    o_ref[0] = jnp.concatenate(sel, axis=1) + cloud * N


def _ball_query_tc(xyzT, nxr, radius, S=1024, interpret=False):
    C, _, N = xyzT.shape
    parts = S // 128
    r2 = radius * radius
    out = pl.pallas_call(
        functools.partial(_bq_body, N, parts, r2),
        grid=(C * parts,),
        in_specs=[
            pl.BlockSpec((1, 3, N), lambda g, parts=parts: (g // parts, 0, 0)),
            pl.BlockSpec((1, 128, 3), lambda g, parts=parts:
                         (g // parts, g % parts, 0)),
        ],
        out_specs=pl.BlockSpec((1, 128, 16), lambda g, parts=parts:
                               (g // parts, g % parts, 0)),
        out_shape=jax.ShapeDtypeStruct((C, S, 16), I32),
        interpret=interpret,
    )(xyzT, nxr.reshape(C, S, 3))
    return out


def _gather_rows_sc(table, gi, D):
    B = gi.shape[0]
    blk_per_w = B // (_NW * 128)
    gi2 = gi.reshape(B // 128, 128)
    mesh = plsc.VectorSubcoreMesh(core_axis_name="c", subcore_axis_name="s")

    @functools.partial(
        pl.kernel, mesh=mesh,
        out_type=jax.ShapeDtypeStruct((B, D), F32),
        scratch_types=[
            pltpu.VMEM((blk_per_w, 128), I32),
            pltpu.VMEM((128, D), F32),
            pltpu.SemaphoreType.DMA,
        ],
    )
    def g(tab_hbm, gi_hbm, out_hbm, gi_v, rows_v, sem):
        cid = lax.axis_index("c")
        sid = lax.axis_index("s")
        wid = sid * _NC + cid
        rowblk = wid * blk_per_w
        pltpu.sync_copy(gi_hbm.at[pl.ds(rowblk, blk_per_w)], gi_v)

        def chunk(j, _):
            pltpu.async_copy(tab_hbm.at[gi_v.at[j]], rows_v, sem).wait()
            pltpu.sync_copy(rows_v, out_hbm.at[pl.ds((rowblk + j) * 128, 128)])
            return 0

        lax.fori_loop(0, blk_per_w, chunk, 0)

    return g(table, gi2)


def _dot(a, b):
    return lax.dot_general(a, b, (((1,), (0,)), ((), ())),
                           precision=lax.Precision.HIGHEST,
                           preferred_element_type=F32)


def _mlp_body(cout, x_ref, c_ref, w1_ref, w1a_ref, b1_ref, w2_ref, b2_ref,
              w3_ref, b3_ref, o_ref):
    ch1 = w1_ref.shape[-1]
    D = x_ref.shape[-1]
    Cpad = jnp.concatenate([c_ref[0], jnp.zeros((128, D - 3), F32)], axis=1)
    Xr = (x_ref[0].reshape(128, 16, D) - Cpad[:, None, :]).reshape(2048, D)
    h = jnp.maximum(_dot(Xr, w1_ref[0]) + b1_ref[0], 0.0)
    h = jnp.maximum(_dot(h, w2_ref[0]) + b2_ref[0], 0.0)
    h = jnp.maximum(_dot(h, w3_ref[0]) + b3_ref[0], 0.0)
    o_ref[0] = jnp.max(h.reshape(128, 16, cout), axis=1)


def _mlp_pallas(Xg, Cb, w1, w1a, b1, w2, b2, w3, b3, cout):
    D = Xg.shape[-1]
    ch1 = w1.shape[-1]
    ch2 = w2.shape[-1]
    out = pl.pallas_call(
        functools.partial(_mlp_body, cout),
        grid=(32,),
        in_specs=[
            pl.BlockSpec((1, 2048, D), lambda g: (g, 0, 0)),
            pl.BlockSpec((1, 128, 3), lambda g: (g, 0, 0)),
            pl.BlockSpec((1, D, ch1), lambda g: (g // 16, 0, 0)),
            pl.BlockSpec((1, 3, ch1), lambda g: (g // 16, 0, 0)),
            pl.BlockSpec((1, 1, ch1), lambda g: (g // 16, 0, 0)),
            pl.BlockSpec((1, ch1, ch2), lambda g: (g // 16, 0, 0)),
            pl.BlockSpec((1, 1, ch2), lambda g: (g // 16, 0, 0)),
            pl.BlockSpec((1, ch2, cout), lambda g: (g // 16, 0, 0)),
            pl.BlockSpec((1, 1, cout), lambda g: (g // 16, 0, 0)),
        ],
        out_specs=pl.BlockSpec((1, 128, cout), lambda g: (g, 0, 0)),
        out_shape=jax.ShapeDtypeStruct((32, 128, cout), F32),
    )(Xg, Cb, w1, w1a, b1, w2, b2, w3, b3)
    return out


def _stack2(a, b):
    return jnp.stack([a, b])


def _prep_params(p_f1, p_f2, D, nrel):
    def w1full(p):
        W = p[0][0]
        pad = jnp.zeros((D - W.shape[0], W.shape[1]), F32)
        return jnp.concatenate([W, pad], axis=0)
    w1 = _stack2(w1full(p_f1), w1full(p_f2))
    w1a = _stack2(p_f1[0][0][0:nrel], p_f2[0][0][0:nrel])
    b1 = _stack2(p_f1[0][1], p_f2[0][1])[:, None, :]
    w2 = _stack2(p_f1[1][0], p_f2[1][0])
    b2 = _stack2(p_f1[1][1], p_f2[1][1])[:, None, :]
    w3 = _stack2(p_f1[2][0], p_f2[2][0])
    b3 = _stack2(p_f1[2][1], p_f2[2][1])[:, None, :]
    return w1, w1a, b1, w2, b2, w3, b3


def kernel(l0_xyz_f1, l0_xyz_f2, params_sa1_f1, params_sa2_f1, params_sa1_f2, params_sa2_f2):
    xyz = jnp.concatenate([l0_xyz_f1, l0_xyz_f2], axis=0)
    xyzT = jnp.transpose(xyz, (0, 2, 1))

    nx1T = _fps_pallas(xyzT, 1024)
    nxr1 = jnp.transpose(nx1T, (0, 2, 1))
    gi1 = _ball_query_tc(xyzT, nxr1, 0.5).reshape(65536)
    tab1 = jnp.concatenate(
        [xyz.reshape(4 * 16384, 3), jnp.zeros((4 * 16384, 125), F32)], axis=1)
    Xg1 = _gather_rows_sc(tab1, gi1, 128).reshape(32, 2048, 128)
    Cb1 = nxr1.reshape(32, 128, 3)
    w1, w1a, b1, w2, b2, w3, b3 = _prep_params(params_sa1_f1, params_sa1_f2,
                                               128, 3)
    l1p = _mlp_pallas(Xg1, Cb1, w1, w1a, b1, w2, b2, w3, b3, 64)
    l1p = l1p.reshape(4, 1024, 64)

    nx2T = _fps_pallas(nx1T, 1024)
    nxr2 = jnp.transpose(nx2T, (0, 2, 1))
    gi2 = _ball_query_tc(nx1T, nxr2, 1.0).reshape(65536)
    tab2 = jnp.concatenate(
        [nxr1.reshape(4096, 3), l1p.reshape(4096, 64),
         jnp.zeros((4096, 61), F32)],
        axis=1)
    Xg2 = _gather_rows_sc(tab2, gi2, 128).reshape(32, 2048, 128)
    Cb2 = nxr2.reshape(32, 128, 3)
    w1_, w1a_, b1_, w2_, b2_, w3_, b3_ = _prep_params(
        params_sa2_f1, params_sa2_f2, 128, 3)
    l2p = _mlp_pallas(Xg2, Cb2, w1_, w1a_, b1_, w2_, b2_, w3_, b3_, 128)
    l2p = l2p.reshape(4, 1024, 128)

    new_xyz2 = nxr2
    return (new_xyz2[0:2], l2p[0:2], new_xyz2[2:4], l2p[2:4])


# speedup vs baseline: 12.3488x; 1.0561x over previous
import functools

import jax
import jax.numpy as jnp
from jax import lax
from jax.experimental import pallas as pl
from jax.experimental.pallas import tpu as pltpu
from jax.experimental.pallas import tpu_sc as plsc

F32 = jnp.float32
I32 = jnp.int32

_NC = 2
_NS = 16
_NW = _NC * _NS


def _fps_body(C, npoint, R, N, x_ref, o_ref, *dist_refs):
    for c in range(C):
        dist_refs[c][...] = jnp.full((R, 128), 1e10, F32)
    lane = lax.broadcasted_iota(I32, (1, 128), 1)

    def body(step, fars):
        flat = (lax.broadcasted_iota(I32, (R, 128), 0) * 128
                + lax.broadcasted_iota(I32, (R, 128), 1))
        new_fars = []
        for c in range(C):
            far = fars[c]
            r = far // 128
            col = far % 128
            lsel = lane == col
            rx = x_ref[c, 0, pl.ds(r, 1), :]
            ry = x_ref[c, 1, pl.ds(r, 1), :]
            rz = x_ref[c, 2, pl.ds(r, 1), :]
            cx = jnp.sum(jnp.where(lsel, rx, 0.0))
            cy = jnp.sum(jnp.where(lsel, ry, 0.0))
            cz = jnp.sum(jnp.where(lsel, rz, 0.0))
            o_ref[c, 0, step] = cx
            o_ref[c, 1, step] = cy
            o_ref[c, 2, step] = cz
            dx = x_ref[c, 0] - cx
            dy = x_ref[c, 1] - cy
            dz = x_ref[c, 2] - cz
            d = (dx * dx + dz * dz) + dy * dy
            nd = jnp.minimum(dist_refs[c][...], d)
            dist_refs[c][...] = nd
            m = jnp.max(nd)
            new_fars.append(jnp.min(jnp.where(nd == m, flat, N)))
        return tuple(new_fars)

    lax.fori_loop(0, npoint, body, (jnp.int32(0),) * C)


def _fps_pallas(xyz4, npoint, interpret=False):
    C, _, N = xyz4.shape
    R = N // 128
    xr = xyz4.reshape(C, 3, R, 128)
    out = pl.pallas_call(
        functools.partial(_fps_body, C, npoint, R, N),
        grid=(1,),
        in_specs=[pl.BlockSpec((C, 3, R, 128), lambda g: (0, 0, 0, 0))],
        out_specs=pl.BlockSpec((C, 3, npoint), lambda g: (0, 0, 0),
                               memory_space=pltpu.SMEM),
        out_shape=jax.ShapeDtypeStruct((C, 3, npoint), F32),
        scratch_shapes=[pltpu.VMEM((R, 128), F32) for _ in range(C)],
        interpret=interpret,
    )(xr)
    return out


def _bq_body(N, parts, r2, x_ref, c_ref, o_ref):
    cloud = pl.program_id(0) // parts
    X = x_ref[0, 0].reshape(1, N)
    Y = x_ref[0, 1].reshape(1, N)
    Z = x_ref[0, 2].reshape(1, N)
    cx = c_ref[0][:, 0:1]
    cy = c_ref[0][:, 1:2]
    cz = c_ref[0][:, 2:3]
    dx = X - cx
    dy = Y - cy
    dz = Z - cz
    dsq = (dx * dx + dz * dz) + dy * dy
    idx = lax.broadcasted_iota(I32, (1, N), 1)
    M = jnp.where(dsq <= r2, idx, N)
    mins = []
    for _ in range(16):
        mn = jnp.min(M, axis=1, keepdims=True)
        mins.append(mn)
        M = jnp.where(M == mn, N, M)
    first = mins[0]
    sel = [first] + [jnp.where(m == N, first, m) for m in mins[1:]]
    o_ref[0] = jnp.concatenate(sel, axis=1) + cloud * N


def _ball_query_tc(xyzT, nxr, radius, S=1024, interpret=False):
    C, _, N = xyzT.shape
    parts = S // 128
    r2 = radius * radius
    out = pl.pallas_call(
        functools.partial(_bq_body, N, parts, r2),
        grid=(C * parts,),
        in_specs=[
            pl.BlockSpec((1, 3, N), lambda g, parts=parts: (g // parts, 0, 0)),
            pl.BlockSpec((1, 128, 3), lambda g, parts=parts:
                         (g // parts, g % parts, 0)),
        ],
        out_specs=pl.BlockSpec((1, 128, 16), lambda g, parts=parts:
                               (g // parts, g % parts, 0)),
        out_shape=jax.ShapeDtypeStruct((C, S, 16), I32),
        interpret=interpret,
    )(xyzT, nxr.reshape(C, S, 3))
    return out


def _gather_rows_sc(table, gi, D):
    B = gi.shape[0]
    blk_per_w = B // (_NW * 128)
    gi2 = gi.reshape(B // 128, 128)
    mesh = plsc.VectorSubcoreMesh(core_axis_name="c", subcore_axis_name="s")

    @functools.partial(
        pl.kernel, mesh=mesh,
        out_type=jax.ShapeDtypeStruct((B, D), F32),
        scratch_types=[
            pltpu.VMEM((blk_per_w, 128), I32),
            pltpu.VMEM((128, D), F32),
            pltpu.SemaphoreType.DMA,
        ],
    )
    def g(tab_hbm, gi_hbm, out_hbm, gi_v, rows_v, sem):
        cid = lax.axis_index("c")
        sid = lax.axis_index("s")
        wid = sid * _NC + cid
        rowblk = wid * blk_per_w
        pltpu.sync_copy(gi_hbm.at[pl.ds(rowblk, blk_per_w)], gi_v)

        def chunk(j, _):
            pltpu.async_copy(tab_hbm.at[gi_v.at[j]], rows_v, sem).wait()
            pltpu.sync_copy(rows_v, out_hbm.at[pl.ds((rowblk + j) * 128, 128)])
            return 0

        lax.fori_loop(0, blk_per_w, chunk, 0)

    return g(table, gi2)


def _dot(a, b):
    return lax.dot_general(a, b, (((1,), (0,)), ((), ())),
                           precision=lax.Precision.DEFAULT,
                           preferred_element_type=F32)


def _mlp_body(cout, x_ref, c_ref, w1_ref, w1a_ref, b1_ref, w2_ref, b2_ref,
              w3_ref, b3_ref, o_ref):
    ch1 = w1_ref.shape[-1]
    D = x_ref.shape[-1]
    Cpad = jnp.concatenate([c_ref[0], jnp.zeros((128, D - 3), F32)], axis=1)
    Xr = (x_ref[0].reshape(128, 16, D) - Cpad[:, None, :]).reshape(2048, D)
    h = jnp.maximum(_dot(Xr, w1_ref[0]) + b1_ref[0], 0.0)
    h = jnp.maximum(_dot(h, w2_ref[0]) + b2_ref[0], 0.0)
    h = jnp.maximum(_dot(h, w3_ref[0]) + b3_ref[0], 0.0)
    o_ref[0] = jnp.max(h.reshape(128, 16, cout), axis=1)


def _mlp_pallas(Xg, Cb, w1, w1a, b1, w2, b2, w3, b3, cout):
    D = Xg.shape[-1]
    ch1 = w1.shape[-1]
    ch2 = w2.shape[-1]
    out = pl.pallas_call(
        functools.partial(_mlp_body, cout),
        grid=(32,),
        in_specs=[
            pl.BlockSpec((1, 2048, D), lambda g: (g, 0, 0)),
            pl.BlockSpec((1, 128, 3), lambda g: (g, 0, 0)),
            pl.BlockSpec((1, D, ch1), lambda g: (g // 16, 0, 0)),
            pl.BlockSpec((1, 3, ch1), lambda g: (g // 16, 0, 0)),
            pl.BlockSpec((1, 1, ch1), lambda g: (g // 16, 0, 0)),
            pl.BlockSpec((1, ch1, ch2), lambda g: (g // 16, 0, 0)),
            pl.BlockSpec((1, 1, ch2), lambda g: (g // 16, 0, 0)),
            pl.BlockSpec((1, ch2, cout), lambda g: (g // 16, 0, 0)),
            pl.BlockSpec((1, 1, cout), lambda g: (g // 16, 0, 0)),
        ],
        out_specs=pl.BlockSpec((1, 128, cout), lambda g: (g, 0, 0)),
        out_shape=jax.ShapeDtypeStruct((32, 128, cout), F32),
    )(Xg, Cb, w1, w1a, b1, w2, b2, w3, b3)
    return out


def _stack2(a, b):
    return jnp.stack([a, b])


def _prep_params(p_f1, p_f2, D, nrel):
    def w1full(p):
        W = p[0][0]
        pad = jnp.zeros((D - W.shape[0], W.shape[1]), F32)
        return jnp.concatenate([W, pad], axis=0)
    w1 = _stack2(w1full(p_f1), w1full(p_f2))
    w1a = _stack2(p_f1[0][0][0:nrel], p_f2[0][0][0:nrel])
    b1 = _stack2(p_f1[0][1], p_f2[0][1])[:, None, :]
    w2 = _stack2(p_f1[1][0], p_f2[1][0])
    b2 = _stack2(p_f1[1][1], p_f2[1][1])[:, None, :]
    w3 = _stack2(p_f1[2][0], p_f2[2][0])
    b3 = _stack2(p_f1[2][1], p_f2[2][1])[:, None, :]
    return w1, w1a, b1, w2, b2, w3, b3


def kernel(l0_xyz_f1, l0_xyz_f2, params_sa1_f1, params_sa2_f1, params_sa1_f2, params_sa2_f2):
    xyz = jnp.concatenate([l0_xyz_f1, l0_xyz_f2], axis=0)
    xyzT = jnp.transpose(xyz, (0, 2, 1))

    nx1T = _fps_pallas(xyzT, 1024)
    nxr1 = jnp.transpose(nx1T, (0, 2, 1))
    gi1 = _ball_query_tc(xyzT, nxr1, 0.5).reshape(65536)
    tab1 = jnp.concatenate(
        [xyz.reshape(4 * 16384, 3), jnp.zeros((4 * 16384, 125), F32)], axis=1)
    Xg1 = _gather_rows_sc(tab1, gi1, 128).reshape(32, 2048, 128)
    Cb1 = nxr1.reshape(32, 128, 3)
    w1, w1a, b1, w2, b2, w3, b3 = _prep_params(params_sa1_f1, params_sa1_f2,
                                               128, 3)
    l1p = _mlp_pallas(Xg1, Cb1, w1, w1a, b1, w2, b2, w3, b3, 64)
    l1p = l1p.reshape(4, 1024, 64)

    nx2T = _fps_pallas(nx1T, 1024)
    nxr2 = jnp.transpose(nx2T, (0, 2, 1))
    gi2 = _ball_query_tc(nx1T, nxr2, 1.0).reshape(65536)
    tab2 = jnp.concatenate(
        [nxr1.reshape(4096, 3), l1p.reshape(4096, 64),
         jnp.zeros((4096, 61), F32)],
        axis=1)
    Xg2 = _gather_rows_sc(tab2, gi2, 128).reshape(32, 2048, 128)
    Cb2 = nxr2.reshape(32, 128, 3)
    w1_, w1a_, b1_, w2_, b2_, w3_, b3_ = _prep_params(
        params_sa2_f1, params_sa2_f2, 128, 3)
    l2p = _mlp_pallas(Xg2, Cb2, w1_, w1a_, b1_, w2_, b2_, w3_, b3_, 128)
    l2p = l2p.reshape(4, 1024, 128)

    new_xyz2 = nxr2
    return (new_xyz2[0:2], l2p[0:2], new_xyz2[2:4], l2p[2:4])
